# baseline (speedup 1.0000x reference)
import jax
import jax.numpy as jnp
from jax.experimental import pallas as pl

F = 100
SEG = 64
B = 4096
D = F * SEG
RED = 50
TILE_B = 128


def _body(x_ref, w1_ref, w2_ref, o_ref):
    xb = x_ref[...]
    xx = jnp.sum(xb, axis=2) * (1.0 / SEG)
    h = jax.lax.dot_general(xx, w1_ref[...], (((1,), (1,)), ((), ())),
                            preferred_element_type=jnp.float32)
    h = jnp.maximum(h, 0.0)
    s = jax.lax.dot_general(h, w2_ref[...], (((1,), (1,)), ((), ())),
                            preferred_element_type=jnp.float32)
    s = jax.nn.sigmoid(s)
    o_ref[...] = xb * s[:, :, None]


def kernel(x, W1, W2):
    x3 = x.reshape(B, F, SEG)
    out3 = pl.pallas_call(
        _body,
        grid=(B // TILE_B,),
        in_specs=[
            pl.BlockSpec((TILE_B, F, SEG), lambda i: (i, 0, 0)),
            pl.BlockSpec((RED, F), lambda i: (0, 0)),
            pl.BlockSpec((F, RED), lambda i: (0, 0)),
        ],
        out_specs=pl.BlockSpec((TILE_B, F, SEG), lambda i: (i, 0, 0)),
        out_shape=jax.ShapeDtypeStruct((B, F, SEG), jnp.float32),
    )(x3, W1, W2)
    return out3.reshape(B, D)


# speedup vs baseline: 7.4134x; 7.4134x over previous
import numpy as np
import jax
import jax.numpy as jnp
from jax.experimental import pallas as pl

F = 100
SEG = 64
B = 4096
D = F * SEG
RED = 50
TILE_B = 256

_SM = np.repeat(np.eye(F, dtype=np.float32), SEG, axis=0) * (1.0 / SEG)
_RM = np.repeat(np.eye(F, dtype=np.float32), SEG, axis=1)


def _body(x_ref, w1t_ref, w2t_ref, sm_ref, rm_ref, o_ref):
    xb = x_ref[...]
    x_hi = xb.astype(jnp.bfloat16)
    x_lo = (xb - x_hi.astype(jnp.float32)).astype(jnp.bfloat16)
    sm = sm_ref[...]
    xx = (jnp.dot(x_hi, sm, preferred_element_type=jnp.float32)
          + jnp.dot(x_lo, sm, preferred_element_type=jnp.float32))
    h = jnp.maximum(jnp.dot(xx, w1t_ref[...],
                            preferred_element_type=jnp.float32), 0.0)
    s = jax.nn.sigmoid(jnp.dot(h, w2t_ref[...],
                               preferred_element_type=jnp.float32))
    s_hi = s.astype(jnp.bfloat16)
    s_lo = (s - s_hi.astype(jnp.float32)).astype(jnp.bfloat16)
    rm = rm_ref[...]
    s_rep = (jnp.dot(s_hi, rm, preferred_element_type=jnp.float32)
             + jnp.dot(s_lo, rm, preferred_element_type=jnp.float32))
    o_ref[...] = xb * s_rep


def kernel(x, W1, W2):
    w1t = W1.T
    w2t = W2.T
    sm = jnp.asarray(_SM, dtype=jnp.bfloat16)
    rm = jnp.asarray(_RM, dtype=jnp.bfloat16)
    return pl.pallas_call(
        _body,
        grid=(B // TILE_B,),
        in_specs=[
            pl.BlockSpec((TILE_B, D), lambda i: (i, 0)),
            pl.BlockSpec((F, RED), lambda i: (0, 0)),
            pl.BlockSpec((RED, F), lambda i: (0, 0)),
            pl.BlockSpec((D, F), lambda i: (0, 0)),
            pl.BlockSpec((F, D), lambda i: (0, 0)),
        ],
        out_specs=pl.BlockSpec((TILE_B, D), lambda i: (i, 0)),
        out_shape=jax.ShapeDtypeStruct((B, D), jnp.float32),
    )(x, w1t, w2t, sm, rm)
